# Initial kernel scaffold; baseline (speedup 1.0000x reference)
import jax
import jax.numpy as jnp
from jax.experimental import pallas as pl


def kernel(inp_ids, inp_num, ns0, ns1, ids0, num0, ids1, num1, W0, W1, decay0, decay1, thr0, thr1):
    raise NotImplementedError("write your pallas kernel here")


# speedup vs baseline: 1.4780x; 1.4780x over previous
import functools

import jax
import jax.numpy as jnp
from jax.experimental import pallas as pl
from jax.experimental.pallas import tpu as pltpu

D = 2048
SS = 128
BB = 64
C = 2 * BB


def _cgt(va, ia, vb, ib):
    return (va > vb) | ((va == vb) & (ia < ib))


def _cx(v, i, size, stride, flip):
    R, Cc = v.shape
    pos = jax.lax.broadcasted_iota(jnp.int32, (R, Cc), 0)
    if stride >= 8:
        m = R // (2 * stride)
        v4 = v.reshape(m, 2, stride, Cc)
        i4 = i.reshape(m, 2, stride, Cc)
        dirm = ((pos.reshape(m, 2, stride, Cc)[:, 0] & size) == 0) ^ flip
        a_v, b_v = v4[:, 0], v4[:, 1]
        a_i, b_i = i4[:, 0], i4[:, 1]
        gt = _cgt(a_v, a_i, b_v, b_i)
        first = ~(dirm ^ gt)
        lo_v = jnp.where(first, a_v, b_v)
        hi_v = jnp.where(first, b_v, a_v)
        lo_i = jnp.where(first, a_i, b_i)
        hi_i = jnp.where(first, b_i, a_i)
        v = jnp.stack([lo_v, hi_v], axis=1).reshape(R, Cc)
        i = jnp.stack([lo_i, hi_i], axis=1).reshape(R, Cc)
    else:
        desc = ((pos & size) == 0) ^ flip
        low = (pos & stride) == 0
        pv = jnp.where(low, jnp.roll(v, -stride, axis=0), jnp.roll(v, stride, axis=0))
        pi = jnp.where(low, jnp.roll(i, -stride, axis=0), jnp.roll(i, stride, axis=0))
        gt = _cgt(v, i, pv, pi)
        first = ~(desc ^ gt)
        take = ~(low ^ first)
        v = jnp.where(take, v, pv)
        i = jnp.where(take, i, pi)
    return v, i


def _lif_body(num_sref, ids_sref, wt0, wt1, ns_ref, dec_ref, thr_ref,
              nns_ref, diff_ref, acc_ref):
    l = pl.program_id(0)
    b = pl.program_id(1)
    n = jnp.clip(num_sref[l, b], 0, SS)
    acc_ref[...] = jnp.zeros((8, 256), jnp.float32)

    def mk_body(wref):
        def body(j, carry):
            c = ids_sref[l, b, j]
            acc_ref[...] += wref[c]
            return carry
        return body

    @pl.when(l == 0)
    def _():
        jax.lax.fori_loop(0, n, mk_body(wt0), 0)

    @pl.when(l != 0)
    def _():
        jax.lax.fori_loop(0, n, mk_body(wt1), 0)

    syn = acc_ref[...]
    ns = ns_ref[0, 0]
    dec = dec_ref[l]
    th = thr_ref[l]
    ns_reset = jnp.where(ns >= th, 0.0, ns)
    nns = ns_reset * dec + (1.0 - dec) * syn
    nns_ref[0, 0] = nns
    diff_ref[0, 0] = nns - th


def _sort_blocks_body(x_ref, v_ref, i_ref, cnt_ref):
    g = pl.program_id(0)
    v = x_ref[...]
    i = jax.lax.broadcasted_iota(jnp.int32, (SS, C), 0) + g * SS
    cnt_ref[0, 0] = jnp.sum(jnp.where(v >= 0.0, jnp.int32(1), jnp.int32(0)),
                            axis=0)
    size = 2
    while size < SS:
        s = size // 2
        while s >= 1:
            v, i = _cx(v, i, size, s, False)
            s //= 2
        size *= 2
    flip = (g & 1) == 1
    for s in (64, 32, 16, 8, 4, 2, 1):
        v, i = _cx(v, i, SS, s, flip)
    v_ref[...] = v
    i_ref[...] = i


def _merge_body(va_ref, ia_ref, v_ref, i_ref):
    g = pl.program_id(0)
    a_v, b_v = va_ref[0:SS, :], va_ref[SS:2 * SS, :]
    a_i, b_i = ia_ref[0:SS, :], ia_ref[SS:2 * SS, :]
    gt = _cgt(a_v, a_i, b_v, b_i)
    v = jnp.where(gt, a_v, b_v)
    i = jnp.where(gt, a_i, b_i)
    flip = (g & 1) == 1
    for s in (64, 32, 16, 8, 4, 2, 1):
        v, i = _cx(v, i, SS, s, flip)
    v_ref[...] = v
    i_ref[...] = i


def _final_body(va_ref, ia_ref, cnt_ref, ids_ref, num_ref):
    a_v, b_v = va_ref[0:SS, :], va_ref[SS:2 * SS, :]
    a_i, b_i = ia_ref[0:SS, :], ia_ref[SS:2 * SS, :]
    gt = _cgt(a_v, a_i, b_v, b_i)
    v = jnp.where(gt, a_v, b_v)
    i = jnp.where(gt, a_i, b_i)
    for s in (64, 32, 16, 8, 4, 2, 1):
        v, i = _cx(v, i, SS, s, False)
    ids_ref[...] = i.astype(jnp.float32)
    cnt = jnp.sum(cnt_ref[...], axis=0)
    num_ref[...] = jnp.broadcast_to(jnp.minimum(cnt, SS), (8, C))


@functools.partial(jax.jit, static_argnames=("interpret",))
def _impl(ids, num, wt0, wt1, ns, dec, th, interpret=False):
    lif = pl.pallas_call(
        _lif_body,
        grid=(2, BB),
        in_specs=[
            pl.BlockSpec(memory_space=pltpu.SMEM),
            pl.BlockSpec(memory_space=pltpu.SMEM),
            pl.BlockSpec(memory_space=pltpu.VMEM),
            pl.BlockSpec(memory_space=pltpu.VMEM),
            pl.BlockSpec((1, 1, 8, 256), lambda l, b: (l, b, 0, 0)),
            pl.BlockSpec(memory_space=pltpu.VMEM),
            pl.BlockSpec(memory_space=pltpu.VMEM),
        ],
        out_specs=[
            pl.BlockSpec((1, 1, 8, 256), lambda l, b: (l, b, 0, 0)),
            pl.BlockSpec((1, 1, 8, 256), lambda l, b: (l, b, 0, 0)),
        ],
        out_shape=[
            jax.ShapeDtypeStruct((2, BB, 8, 256), jnp.float32),
            jax.ShapeDtypeStruct((2, BB, 8, 256), jnp.float32),
        ],
        scratch_shapes=[pltpu.VMEM((8, 256), jnp.float32)],
        interpret=interpret,
    )
    nns, diff = lif(num, ids, wt0, wt1, ns, dec, th)
    diff_t = diff.reshape(C, D).T

    nb = D // SS
    sv, si, cnt = pl.pallas_call(
        _sort_blocks_body,
        grid=(nb,),
        in_specs=[pl.BlockSpec((SS, C), lambda g: (g, 0))],
        out_specs=[
            pl.BlockSpec((SS, C), lambda g: (g, 0)),
            pl.BlockSpec((SS, C), lambda g: (g, 0)),
            pl.BlockSpec((1, 1, C), lambda g: (g, 0, 0)),
        ],
        out_shape=[
            jax.ShapeDtypeStruct((D, C), jnp.float32),
            jax.ShapeDtypeStruct((D, C), jnp.int32),
            jax.ShapeDtypeStruct((nb, 1, C), jnp.int32),
        ],
        interpret=interpret,
    )(diff_t)

    r = D
    while r > 2 * SS:
        sv, si = pl.pallas_call(
            _merge_body,
            grid=(r // (2 * SS),),
            in_specs=[
                pl.BlockSpec((2 * SS, C), lambda g: (g, 0)),
                pl.BlockSpec((2 * SS, C), lambda g: (g, 0)),
            ],
            out_specs=[
                pl.BlockSpec((SS, C), lambda g: (g, 0)),
                pl.BlockSpec((SS, C), lambda g: (g, 0)),
            ],
            out_shape=[
                jax.ShapeDtypeStruct((r // 2, C), jnp.float32),
                jax.ShapeDtypeStruct((r // 2, C), jnp.int32),
            ],
            interpret=interpret,
        )(sv, si)
        r //= 2

    idsf, numo = pl.pallas_call(
        _final_body,
        in_specs=[
            pl.BlockSpec(memory_space=pltpu.VMEM),
            pl.BlockSpec(memory_space=pltpu.VMEM),
            pl.BlockSpec(memory_space=pltpu.VMEM),
        ],
        out_shape=[
            jax.ShapeDtypeStruct((SS, C), jnp.float32),
            jax.ShapeDtypeStruct((8, C), jnp.int32),
        ],
        interpret=interpret,
    )(sv, si, cnt)
    return nns, idsf, numo


def kernel(inp_ids, inp_num, ns0, ns1, ids0, num0, ids1, num1,
           W0, W1, decay0, decay1, thr0, thr1, _interpret=False):
    del ids1, num1
    ids = jnp.stack([inp_ids, ids0]).astype(jnp.int32)
    num = jnp.stack([inp_num[:, 0], num0[:, 0]])
    wt0 = W0.T.reshape(D, 8, 256)
    wt1 = W1.T.reshape(D, 8, 256)
    ns = jnp.stack([ns0, ns1]).reshape(2, BB, 8, 256)
    dec = jnp.stack([decay0, decay1]).reshape(2, 8, 256)
    th = jnp.stack([thr0, thr1]).reshape(2, 8, 256)
    nns, idsf, numo = _impl(ids, num, wt0, wt1, ns, dec, th,
                            interpret=_interpret)
    ids_t = idsf.T
    o0_ids, o1_ids = ids_t[:BB], ids_t[BB:]
    o0_num = numo[0, :BB].reshape(BB, 1)
    o1_num = numo[0, BB:].reshape(BB, 1)
    nns2 = nns.reshape(2, BB, D)
    return (o0_ids, o0_num, o1_ids, o1_num, nns2[0], nns2[1])
